# baseline (speedup 1.0000x reference)
import jax
import jax.numpy as jnp
from jax import lax
from jax.experimental import pallas as pl
from jax.experimental.pallas import tpu as pltpu
from jax.experimental.pallas import tpu_sc as plsc

T = 512
L = 16
NUM_CHARS = 256
NUM_TAGS = 1024
CE = 128
CR = 256
WR = 512
WE = 256
BEAM = 16

_F32 = jnp.float32


def _dense_body(fids_ref, bids_ref, cemb_ref, wxf_ref, whf_ref, bf_ref,
                wxb_ref, whb_ref, bb_ref, wp_ref, bp_ref, wwf_ref, uwf_ref,
                bwf_ref, wwb_ref, uwb_ref, bwb_ref, wo_ref, bo_ref,
                scores_ref, xf_ref, xb_ref):
    RB = 128
    for rb in range(T // RB):
        r0 = rb * RB
        hf = jnp.zeros((RB, CR), _F32)
        hb = jnp.zeros((RB, CR), _F32)
        for t in range(L):
            idf = fids_ref[t, r0:r0 + RB]
            idb = bids_ref[t, r0:r0 + RB]
            ohf = (lax.broadcasted_iota(jnp.int32, (RB, NUM_CHARS), 1)
                   == idf[:, None]).astype(_F32)
            ohb = (lax.broadcasted_iota(jnp.int32, (RB, NUM_CHARS), 1)
                   == idb[:, None]).astype(_F32)
            ef = jnp.dot(ohf, cemb_ref[...], precision=lax.Precision.HIGHEST,
                         preferred_element_type=_F32)
            eb = jnp.dot(ohb, cemb_ref[...], precision=lax.Precision.HIGHEST,
                         preferred_element_type=_F32)
            xf = jnp.dot(ef, wxf_ref[...], preferred_element_type=_F32)
            xb = jnp.dot(eb, wxb_ref[...], preferred_element_type=_F32)
            hf = jnp.tanh((xf + jnp.dot(hf, whf_ref[...],
                                        preferred_element_type=_F32))
                          + bf_ref[...])
            hb = jnp.tanh((xb + jnp.dot(hb, whb_ref[...],
                                        preferred_element_type=_F32))
                          + bb_ref[...])
        w_in = jnp.tanh(jnp.dot(jnp.concatenate([hf, hb], axis=-1),
                                wp_ref[...], preferred_element_type=_F32)
                        + bp_ref[...])
        xf_ref[r0:r0 + RB, :] = jnp.dot(w_in, wwf_ref[...],
                                        preferred_element_type=_F32)
        xb_ref[r0:r0 + RB, :] = jnp.dot(w_in, wwb_ref[...],
                                        preferred_element_type=_F32)

    def wstep_f(t, hfv):
        zf = jnp.dot(hfv, uwf_ref[...], preferred_element_type=_F32)
        hfv = jnp.tanh((xf_ref[t, :][None, :] + zf) + bwf_ref[...])
        xf_ref[t, :] = hfv[0, :]
        return hfv

    def wstep_b(t, hbv):
        s = T - 1 - t
        zb = jnp.dot(hbv, uwb_ref[...], preferred_element_type=_F32)
        hbv = jnp.tanh((xb_ref[s, :][None, :] + zb) + bwb_ref[...])
        xb_ref[s, :] = hbv[0, :]
        return hbv

    lax.fori_loop(0, T, wstep_f, jnp.zeros((1, WR), _F32))
    lax.fori_loop(0, T, wstep_b, jnp.zeros((1, WR), _F32))

    for rb in range(T // RB):
        r0 = rb * RB
        cat = jnp.concatenate([xf_ref[r0:r0 + RB, :],
                               xb_ref[r0:r0 + RB, :]], axis=-1)
        scores_ref[r0:r0 + RB, :] = (
            jnp.dot(cat, wo_ref[...], preferred_element_type=_F32)
            + bo_ref[...])


def _dense_scores(fidsT, bidsT, cemb, wxf, whf, bf, wxb, whb, bb, wp, bp2,
                  wwf, uwf, bwf, wwb, uwb, bwb, wo, bo2, interpret=False):
    return pl.pallas_call(
        _dense_body,
        out_shape=jax.ShapeDtypeStruct((T, NUM_TAGS), _F32),
        scratch_shapes=[
            pltpu.VMEM((T, WR), _F32),
            pltpu.VMEM((T, WR), _F32),
        ],
        interpret=interpret,
    )(fidsT, bidsT, cemb, wxf, whf, bf, wxb, whb, bb, wp, bp2,
      wwf, uwf, bwf, wwb, uwb, bwb, wo, bo2)


_N_TILES = 16
_ROWS_PER_TILE = T // _N_TILES
_STEPS_PER_TILE = T // _N_TILES
_LN2 = 0.6931471805599453


def _iota16():
    return lax.broadcasted_iota(jnp.int32, (16,), 0)


def _tile_id():
    return lax.axis_index("s")


def _splat_lane(v, i):
    idx = jnp.full((16,), i, jnp.int32)
    return v.at[idx].get(mode="promise_in_bounds")


def _indirect_gather(src_hbm, idx_ref, idx_off, dst_slice, sem):
    return pltpu.async_copy(src_hbm.at[idx_ref.at[pl.ds(idx_off, 128)]],
                            dst_slice, sem)


def _sc_body(scoresf_hbm, tags_hbm, crfw_hbm, pred_hbm, aux_hbm,
             srow_v, bs_loc, bt_loc, idx_v, wg_v, eg_v, btp_v, tags_v,
             gidx_v, gval_v, wch_v, ech_v, bs_v, bt_v, bp_v, bi_v, pred_v,
             aux_v, scal2_v, sem,
             bs_sh, bt_sh, w_sh, e_sh, scal_sh):
    w = _tile_id()
    it16 = _iota16()

    for batch in range(_ROWS_PER_TILE // 8):
        row0 = w * _ROWS_PER_TILE + batch * 8
        pltpu.sync_copy(scoresf_hbm.at[pl.ds(row0 * NUM_TAGS, 8 * NUM_TAGS)],
                        srow_v)
        carry = []
        for r in range(8):
            v0 = srow_v[pl.ds(r * NUM_TAGS, 16)]
            k0, t0 = plsc.sort_key_val(v0, it16)
            carry.extend([k0, t0])

        def tk_step(c, carry):
            out = []
            ci = c * 16 + it16
            for r in range(8):
                tk, tv = carry[2 * r], carry[2 * r + 1]
                cv = srow_v[pl.ds(r * NUM_TAGS + c * 16, 16)]
                sk, sv = plsc.sort_key_val(cv, ci, descending=True)
                m = sk > tk
                nk = jnp.where(m, sk, tk)
                nv = jnp.where(m, sv, tv)
                nk, nv = plsc.sort_key_val(nk, nv)
                out.extend([nk, nv])
            return tuple(out)

        carry = lax.fori_loop(1, NUM_TAGS // 16, tk_step, tuple(carry))
        for r in range(8):
            tk, tv = carry[2 * r], carry[2 * r + 1]
            bs_loc[batch * 8 + r] = lax.rev(tk, (0,))
            bt_loc[batch * 8 + r] = lax.rev(tv, (0,))
    pltpu.sync_copy(bs_loc, bs_sh.at[pl.ds(w * _ROWS_PER_TILE,
                                           _ROWS_PER_TILE)])
    pltpu.sync_copy(bt_loc, bt_sh.at[pl.ds(w * _ROWS_PER_TILE,
                                           _ROWS_PER_TILE)])
    plsc.subcore_barrier()

    t0 = w * _STEPS_PER_TILE
    pstart = jnp.maximum(t0 - 1, 0)
    off = t0 - pstart
    pltpu.sync_copy(bt_sh.at[pl.ds(pstart, _STEPS_PER_TILE + 1)],
                    btp_v.at[pl.ds(0, _STEPS_PER_TILE + 1)])

    def gidx_step(j, _):
        pidx = jnp.maximum(j - 1 + off, 0)
        cidx = j + off
        pt_row = btp_v[pidx]
        bt_row = btp_v[cidx]
        for i in range(16):
            pt_i = _splat_lane(pt_row, i)
            idx_v[pl.ds(j * 256 + i * 16, 16)] = pt_i * NUM_TAGS + bt_row
        return 0

    lax.fori_loop(0, _STEPS_PER_TILE, gidx_step, 0)
    copies = []
    for g in range(_STEPS_PER_TILE * 2):
        copies.append(_indirect_gather(
            crfw_hbm, idx_v, g * 128,
            wg_v.at[g // 2, pl.ds((g % 2) * 128, 128)], sem))
    for cp in copies:
        cp.wait()

    for q in range(_STEPS_PER_TILE):
        for u in range(16):
            eg_v[q, pl.ds(u * 16, 16)] = jnp.exp(wg_v[q, pl.ds(u * 16, 16)])

    pltpu.sync_copy(wg_v, w_sh.at[pl.ds(t0, _STEPS_PER_TILE)])
    pltpu.sync_copy(eg_v, e_sh.at[pl.ds(t0, _STEPS_PER_TILE)])

    def _sum_gathered(masked):
        acc = jnp.zeros((16,), _F32)
        for c in range(T // 16):
            vv = gval_v[pl.ds(c * 16, 16)]
            if masked and c == T // 16 - 1:
                pos = c * 16 + it16
                vv = jnp.where(pos >= T - 1, jnp.zeros((16,), _F32), vv)
            acc = acc + vv
        tot = jnp.broadcast_to(jnp.sum(acc, axis=0), (16,))
        aux_v[...] = tot

    @pl.when(w == 2)
    def _():
        pltpu.sync_copy(tags_hbm, tags_v)

        def bidx_step(c, _):
            pos = c * 16 + it16
            tv = tags_v[pl.ds(c * 16, 16)]
            gidx_v[pl.ds(c * 16, 16)] = pos * NUM_TAGS + tv
            return 0
        lax.fori_loop(0, T // 16, bidx_step, 0)
        gc = [_indirect_gather(scoresf_hbm, gidx_v, g * 128,
                               gval_v.at[pl.ds(g * 128, 128)], sem)
              for g in range(T // 128)]
        for cp in gc:
            cp.wait()
        _sum_gathered(masked=False)
        pltpu.sync_copy(aux_v, scal_sh.at[0])

    @pl.when(w == 3)
    def _():
        pltpu.sync_copy(tags_hbm, tags_v)

        def cidx_step(c, _):
            pos = c * 16 + it16
            cur = tags_v[pl.ds(c * 16, 16)]
            nxt = plsc.load_gather(tags_v, [jnp.minimum(pos + 1, T - 1)])
            gidx_v[pl.ds(c * 16, 16)] = cur * NUM_TAGS + nxt
            return 0
        lax.fori_loop(0, T // 16, cidx_step, 0)
        gc = [_indirect_gather(crfw_hbm, gidx_v, g * 128,
                               gval_v.at[pl.ds(g * 128, 128)], sem)
              for g in range(T // 128)]
        for cp in gc:
            cp.wait()
        _sum_gathered(masked=True)
        pltpu.sync_copy(aux_v, scal_sh.at[1])

    plsc.subcore_barrier()

    @pl.when(w == _N_TILES - 1)
    def _():
        pltpu.sync_copy(bs_sh, bs_v)
        pltpu.sync_copy(bt_sh, bt_v)
        vs = bs_v[0]
        p = jnp.exp(bs_v[0])
        eacc = jnp.zeros((16,), _F32)
        carry = (vs, p, eacc)
        for k in range(T // 64):
            pltpu.sync_copy(w_sh.at[pl.ds(k * 64, 64)], wch_v)
            pltpu.sync_copy(e_sh.at[pl.ds(k * 64, 64)], ech_v)

            def vfstep(jj, carry, k=k):
                vs, p, eacc = carry
                s = k * 64 + jj
                bsrow = bs_v[s]
                bp = jnp.zeros((16,), jnp.int32)
                nvs = (_splat_lane(vs, 0) + bsrow) + wch_v[jj, pl.ds(0, 16)]
                for i in range(1, 16):
                    vv = (_splat_lane(vs, i) + bsrow) + \
                        wch_v[jj, pl.ds(i * 16, 16)]
                    m = vv > nvs
                    bp = jnp.where(m, jnp.full((16,), i, jnp.int32), bp)
                    nvs = jnp.where(m, vv, nvs)
                bp_v[s] = bp
                acc = _splat_lane(p, 0) * ech_v[jj, pl.ds(0, 16)]
                for i in range(1, 16):
                    acc = acc + _splat_lane(p, i) * \
                        ech_v[jj, pl.ds(i * 16, 16)]
                np_ = acc * jnp.exp(bsrow)
                bits = plsc.bitcast(np_, jnp.int32)
                ex = (bits >> 23) & 0xFF
                emax = jnp.broadcast_to(jnp.max(ex, axis=0), (16,))
                scale = plsc.bitcast((254 - emax) << 23, _F32)
                return nvs, np_ * scale, eacc + (emax - 127).astype(_F32)

            carry = lax.fori_loop(1 if k == 0 else 0, 64, vfstep, carry)
        vs, p, eacc = carry

        mx = jnp.broadcast_to(jnp.max(vs, axis=0), (16,))
        idx = jnp.broadcast_to(
            plsc.all_reduce_ffs(vs == mx), (16,)).astype(jnp.int32)
        bi_v[T - 1] = idx

        def bstep(q, idx):
            s = (T - 1) - q
            row = bp_v[s]
            ni = row.at[idx].get(mode="promise_in_bounds")
            bi_v[s - 1] = ni
            return ni

        lax.fori_loop(0, T - 1, bstep, idx)

        zeros16 = jnp.zeros((16,), jnp.int32)
        for c in range(T // 16):
            pos = c * 16 + it16
            bi_lane = plsc.load_gather(bi_v, [pos, zeros16])
            pred = plsc.load_gather(bt_v, [pos, bi_lane])
            pred_v[pl.ds(c * 16, 16)] = pred
        pltpu.sync_copy(pred_v, pred_hbm)

        s_tot = jnp.broadcast_to(jnp.sum(p, axis=0), (16,))
        bits = plsc.bitcast(s_tot, jnp.int32)
        e = (((bits >> 23) & 0xFF) - 127).astype(_F32)
        m = plsc.bitcast((bits & 0x7FFFFF) | (127 << 23), _F32)
        y = e * _LN2 + 0.7 * (m - 1.0)
        for _ in range(4):
            y = (y - 1.0) + s_tot * jnp.exp(-y)
        logZ = y + eacc * _LN2
        pltpu.sync_copy(scal_sh, scal2_v)
        base_s = scal2_v[0]
        crf_s = scal2_v[1]
        aux_v[...] = -((base_s + crf_s) - logZ)
        pltpu.sync_copy(aux_v, aux_hbm)


def _sc_crf(scores_flat, tags, crfw_flat, interpret=False):
    mesh = plsc.VectorSubcoreMesh(core_axis_name="c", subcore_axis_name="s",
                                  num_cores=1, num_subcores=_N_TILES)
    f = pl.kernel(
        _sc_body,
        out_type=[jax.ShapeDtypeStruct((T,), jnp.int32),
                  jax.ShapeDtypeStruct((16,), _F32)],
        mesh=mesh,
        compiler_params=pltpu.CompilerParams(needs_layout_passes=False,
                                             use_tc_tiling_on_sc=False),
        scratch_types=[
            pltpu.VMEM((8 * NUM_TAGS,), _F32),
            pltpu.VMEM((_ROWS_PER_TILE, 16), _F32),
            pltpu.VMEM((_ROWS_PER_TILE, 16), jnp.int32),
            pltpu.VMEM((_STEPS_PER_TILE * 256,), jnp.int32),
            pltpu.VMEM((_STEPS_PER_TILE, 256), _F32),
            pltpu.VMEM((_STEPS_PER_TILE, 256), _F32),
            pltpu.VMEM((_STEPS_PER_TILE + 8, 16), jnp.int32),
            pltpu.VMEM((T,), jnp.int32),
            pltpu.VMEM((T,), jnp.int32),
            pltpu.VMEM((T,), _F32),
            pltpu.VMEM((64, 256), _F32),
            pltpu.VMEM((64, 256), _F32),
            pltpu.VMEM((T, 16), _F32),
            pltpu.VMEM((T, 16), jnp.int32),
            pltpu.VMEM((T, 16), jnp.int32),
            pltpu.VMEM((T, 16), jnp.int32),
            pltpu.VMEM((T,), jnp.int32),
            pltpu.VMEM((16,), _F32),
            pltpu.VMEM((8, 16), _F32),
            pltpu.SemaphoreType.DMA,
            pltpu.VMEM_SHARED((T, 16), _F32),
            pltpu.VMEM_SHARED((T, 16), jnp.int32),
            pltpu.VMEM_SHARED((T, 256), _F32),
            pltpu.VMEM_SHARED((T, 256), _F32),
            pltpu.VMEM_SHARED((8, 16), _F32),
        ],
        interpret=interpret,
    )
    scores_flat = pltpu.with_memory_space_constraint(
        scores_flat, pltpu.MemorySpace.HBM)
    tags = pltpu.with_memory_space_constraint(tags, pltpu.MemorySpace.HBM)
    crfw_flat = pltpu.with_memory_space_constraint(
        crfw_flat, pltpu.MemorySpace.HBM)
    return f(scores_flat, tags, crfw_flat)


def kernel(fwd_charIDs, bwd_charIDs, tags, C_emb, Wxf, Whf, bf, Wxb, Whb, bb,
           Wp, bp, Wwf, Uwf, bwf, Wwb, Uwb, bwb, Wo, bo, crf_w):
    fidsT = jnp.transpose(fwd_charIDs).astype(jnp.int32)
    bidsT = jnp.transpose(bwd_charIDs).astype(jnp.int32)
    scores = _dense_scores(fidsT, bidsT, C_emb, Wxf, Whf, bf[None, :],
                           Wxb, Whb, bb[None, :], Wp, bp[None, :],
                           Wwf, Uwf, bwf[None, :], Wwb, Uwb, bwb[None, :],
                           Wo, bo[None, :])
    pred, aux = _sc_crf(scores.reshape(-1), tags.astype(jnp.int32),
                        crf_w.reshape(-1))
    return pred, aux[0]


# speedup vs baseline: 1.3320x; 1.3320x over previous
import jax
import jax.numpy as jnp
from jax import lax
from jax.experimental import pallas as pl
from jax.experimental.pallas import tpu as pltpu
from jax.experimental.pallas import tpu_sc as plsc

T = 512
L = 16
NUM_CHARS = 256
NUM_TAGS = 1024
CE = 128
CR = 256
WR = 512
WE = 256
BEAM = 16

_F32 = jnp.float32


def _dense_body(fids_ref, bids_ref, cemb_ref, wxf_ref, whf_ref, bf_ref,
                wxb_ref, whb_ref, bb_ref, wp_ref, bp_ref, wwf_ref, uwf_ref,
                bwf_ref, wwb_ref, uwb_ref, bwb_ref, wo_ref, bo_ref,
                scores_ref, xf_ref, xb_ref):
    RB = 128
    for rb in range(T // RB):
        r0 = rb * RB
        hf = jnp.zeros((RB, CR), _F32)
        hb = jnp.zeros((RB, CR), _F32)
        for t in range(L):
            idf = fids_ref[t, r0:r0 + RB]
            idb = bids_ref[t, r0:r0 + RB]
            ohf = (lax.broadcasted_iota(jnp.int32, (RB, NUM_CHARS), 1)
                   == idf[:, None]).astype(_F32)
            ohb = (lax.broadcasted_iota(jnp.int32, (RB, NUM_CHARS), 1)
                   == idb[:, None]).astype(_F32)
            def _egather(oh):
                hi = jnp.dot(oh, cemb_ref[0], preferred_element_type=_F32)
                mid = jnp.dot(oh, cemb_ref[1], preferred_element_type=_F32)
                lo = jnp.dot(oh, cemb_ref[2], preferred_element_type=_F32)
                return (hi + mid) + lo
            ef = _egather(ohf)
            eb = _egather(ohb)
            xf = jnp.dot(ef, wxf_ref[...], preferred_element_type=_F32)
            xb = jnp.dot(eb, wxb_ref[...], preferred_element_type=_F32)
            hf = jnp.tanh((xf + jnp.dot(hf, whf_ref[...],
                                        preferred_element_type=_F32))
                          + bf_ref[...])
            hb = jnp.tanh((xb + jnp.dot(hb, whb_ref[...],
                                        preferred_element_type=_F32))
                          + bb_ref[...])
        w_in = jnp.tanh(jnp.dot(jnp.concatenate([hf, hb], axis=-1),
                                wp_ref[...], preferred_element_type=_F32)
                        + bp_ref[...])
        xf_ref[r0:r0 + RB, :] = jnp.dot(w_in, wwf_ref[...],
                                        preferred_element_type=_F32)
        xb_ref[r0:r0 + RB, :] = jnp.dot(w_in, wwb_ref[...],
                                        preferred_element_type=_F32)

    def wstep(t, carry):
        hfv, hbv = carry
        s = T - 1 - t
        zf = jnp.dot(hfv, uwf_ref[...], preferred_element_type=_F32)
        zb = jnp.dot(hbv, uwb_ref[...], preferred_element_type=_F32)
        hfv = jnp.tanh((xf_ref[t, :][None, :] + zf) + bwf_ref[...])
        hbv = jnp.tanh((xb_ref[s, :][None, :] + zb) + bwb_ref[...])
        xf_ref[t, :] = hfv[0, :]
        xb_ref[s, :] = hbv[0, :]
        return hfv, hbv
    lax.fori_loop(0, T, wstep,
                  (jnp.zeros((1, WR), _F32), jnp.zeros((1, WR), _F32)))

    for rb in range(T // RB):
        r0 = rb * RB
        cat = jnp.concatenate([xf_ref[r0:r0 + RB, :],
                               xb_ref[r0:r0 + RB, :]], axis=-1)
        scores_ref[r0:r0 + RB, :] = (
            jnp.dot(cat, wo_ref[...], preferred_element_type=_F32)
            + bo_ref[...])


def _dense_scores(fidsT, bidsT, cemb, wxf, whf, bf, wxb, whb, bb, wp, bp2,
                  wwf, uwf, bwf, wwb, uwb, bwb, wo, bo2, interpret=False):
    return pl.pallas_call(
        _dense_body,
        out_shape=jax.ShapeDtypeStruct((T, NUM_TAGS), _F32),
        scratch_shapes=[
            pltpu.VMEM((T, WR), _F32),
            pltpu.VMEM((T, WR), _F32),
        ],
        interpret=interpret,
    )(fidsT, bidsT, cemb, wxf, whf, bf, wxb, whb, bb, wp, bp2,
      wwf, uwf, bwf, wwb, uwb, bwb, wo, bo2)


_N_TILES = 16
_ROWS_PER_TILE = T // _N_TILES
_STEPS_PER_TILE = T // _N_TILES
_LN2 = 0.6931471805599453


def _iota16():
    return lax.broadcasted_iota(jnp.int32, (16,), 0)


def _tile_id():
    return lax.axis_index("s")


def _splat_lane(v, i):
    idx = jnp.full((16,), i, jnp.int32)
    return v.at[idx].get(mode="promise_in_bounds")


def _indirect_gather(src_hbm, idx_ref, idx_off, dst_slice, sem):
    return pltpu.async_copy(src_hbm.at[idx_ref.at[pl.ds(idx_off, 128)]],
                            dst_slice, sem)


def _sc_body(scoresf_hbm, tags_hbm, crfw_hbm, pred_hbm, aux_hbm,
             srow_v, bs_loc, bt_loc, idx_v, wg_v, eg_v, btp_v, tags_v,
             gidx_v, gval_v, wch_v, ech_v, bs_v, bt_v, bp_v, bi_v, pred_v,
             aux_v, scal2_v, sem,
             bs_sh, bt_sh, w_sh, e_sh, scal_sh):
    w = _tile_id()
    it16 = _iota16()

    for batch in range(_ROWS_PER_TILE // 8):
        row0 = w * _ROWS_PER_TILE + batch * 8
        pltpu.sync_copy(scoresf_hbm.at[pl.ds(row0 * NUM_TAGS, 8 * NUM_TAGS)],
                        srow_v)
        carry = []
        for r in range(8):
            v0 = srow_v[pl.ds(r * NUM_TAGS, 16)]
            k0, t0 = plsc.sort_key_val(v0, it16)
            carry.extend([k0, t0])

        def tk_step(c, carry):
            out = []
            ci = c * 16 + it16
            for r in range(8):
                tk, tv = carry[2 * r], carry[2 * r + 1]
                cv = srow_v[pl.ds(r * NUM_TAGS + c * 16, 16)]
                sk, sv = plsc.sort_key_val(cv, ci, descending=True)
                m = sk > tk
                nk = jnp.where(m, sk, tk)
                nv = jnp.where(m, sv, tv)
                nk, nv = plsc.sort_key_val(nk, nv)
                out.extend([nk, nv])
            return tuple(out)

        carry = lax.fori_loop(1, NUM_TAGS // 16, tk_step, tuple(carry))
        for r in range(8):
            tk, tv = carry[2 * r], carry[2 * r + 1]
            bs_loc[batch * 8 + r] = lax.rev(tk, (0,))
            bt_loc[batch * 8 + r] = lax.rev(tv, (0,))
    pltpu.sync_copy(bs_loc, bs_sh.at[pl.ds(w * _ROWS_PER_TILE,
                                           _ROWS_PER_TILE)])
    pltpu.sync_copy(bt_loc, bt_sh.at[pl.ds(w * _ROWS_PER_TILE,
                                           _ROWS_PER_TILE)])
    plsc.subcore_barrier()

    t0 = w * _STEPS_PER_TILE
    pstart = jnp.maximum(t0 - 1, 0)
    off = t0 - pstart
    pltpu.sync_copy(bt_sh.at[pl.ds(pstart, _STEPS_PER_TILE + 1)],
                    btp_v.at[pl.ds(0, _STEPS_PER_TILE + 1)])

    def gidx_step(j, _):
        pidx = jnp.maximum(j - 1 + off, 0)
        cidx = j + off
        pt_row = btp_v[pidx]
        bt_row = btp_v[cidx]
        for i in range(16):
            pt_i = _splat_lane(pt_row, i)
            idx_v[pl.ds(j * 256 + i * 16, 16)] = pt_i * NUM_TAGS + bt_row
        return 0

    lax.fori_loop(0, _STEPS_PER_TILE, gidx_step, 0)
    copies = []
    for g in range(_STEPS_PER_TILE * 2):
        copies.append(_indirect_gather(
            crfw_hbm, idx_v, g * 128,
            wg_v.at[g // 2, pl.ds((g % 2) * 128, 128)], sem))
    for cp in copies:
        cp.wait()

    for q in range(_STEPS_PER_TILE):
        for u in range(16):
            eg_v[q, pl.ds(u * 16, 16)] = jnp.exp(wg_v[q, pl.ds(u * 16, 16)])

    pltpu.sync_copy(wg_v, w_sh.at[pl.ds(t0, _STEPS_PER_TILE)])
    pltpu.sync_copy(eg_v, e_sh.at[pl.ds(t0, _STEPS_PER_TILE)])

    def _sum_gathered(masked):
        acc = jnp.zeros((16,), _F32)
        for c in range(T // 16):
            vv = gval_v[pl.ds(c * 16, 16)]
            if masked and c == T // 16 - 1:
                pos = c * 16 + it16
                vv = jnp.where(pos >= T - 1, jnp.zeros((16,), _F32), vv)
            acc = acc + vv
        tot = jnp.broadcast_to(jnp.sum(acc, axis=0), (16,))
        aux_v[...] = tot

    @pl.when(w == 2)
    def _():
        pltpu.sync_copy(tags_hbm, tags_v)

        def bidx_step(c, _):
            pos = c * 16 + it16
            tv = tags_v[pl.ds(c * 16, 16)]
            gidx_v[pl.ds(c * 16, 16)] = pos * NUM_TAGS + tv
            return 0
        lax.fori_loop(0, T // 16, bidx_step, 0)
        gc = [_indirect_gather(scoresf_hbm, gidx_v, g * 128,
                               gval_v.at[pl.ds(g * 128, 128)], sem)
              for g in range(T // 128)]
        for cp in gc:
            cp.wait()
        _sum_gathered(masked=False)
        pltpu.sync_copy(aux_v, scal_sh.at[0])

    @pl.when(w == 3)
    def _():
        pltpu.sync_copy(tags_hbm, tags_v)

        def cidx_step(c, _):
            pos = c * 16 + it16
            cur = tags_v[pl.ds(c * 16, 16)]
            nxt = plsc.load_gather(tags_v, [jnp.minimum(pos + 1, T - 1)])
            gidx_v[pl.ds(c * 16, 16)] = cur * NUM_TAGS + nxt
            return 0
        lax.fori_loop(0, T // 16, cidx_step, 0)
        gc = [_indirect_gather(crfw_hbm, gidx_v, g * 128,
                               gval_v.at[pl.ds(g * 128, 128)], sem)
              for g in range(T // 128)]
        for cp in gc:
            cp.wait()
        _sum_gathered(masked=True)
        pltpu.sync_copy(aux_v, scal_sh.at[1])

    plsc.subcore_barrier()

    @pl.when(w == _N_TILES - 1)
    def _():
        pltpu.sync_copy(bs_sh, bs_v)
        pltpu.sync_copy(bt_sh, bt_v)
        vs = bs_v[0]
        p = jnp.exp(bs_v[0])
        eacc = jnp.zeros((16,), _F32)
        carry = (vs, p, eacc)
        for k in range(T // 64):
            pltpu.sync_copy(w_sh.at[pl.ds(k * 64, 64)], wch_v)
            pltpu.sync_copy(e_sh.at[pl.ds(k * 64, 64)], ech_v)

            def vfstep(jj, carry, k=k):
                vs, p, eacc = carry
                s = k * 64 + jj
                bsrow = bs_v[s]
                vvs = [(_splat_lane(vs, i) + bsrow)
                       + wch_v[jj, pl.ds(i * 16, 16)] for i in range(16)]
                bps = [jnp.full((16,), i, jnp.int32) for i in range(16)]
                while len(vvs) > 1:
                    nv, nb = [], []
                    for a in range(0, len(vvs), 2):
                        m = vvs[a + 1] > vvs[a]
                        nv.append(jnp.where(m, vvs[a + 1], vvs[a]))
                        nb.append(jnp.where(m, bps[a + 1], bps[a]))
                    vvs, bps = nv, nb
                bp_v[s] = bps[0]
                ts = [_splat_lane(p, i) * ech_v[jj, pl.ds(i * 16, 16)]
                      for i in range(16)]
                while len(ts) > 1:
                    ts = [ts[a] + ts[a + 1] for a in range(0, len(ts), 2)]
                np_ = ts[0] * jnp.exp(bsrow)
                bits = plsc.bitcast(np_, jnp.int32)
                ex = (bits >> 23) & 0xFF
                emax = jnp.broadcast_to(jnp.max(ex, axis=0), (16,))
                scale = plsc.bitcast((254 - emax) << 23, _F32)
                return (vvs[0], np_ * scale,
                        eacc + (emax - 127).astype(_F32))

            carry = lax.fori_loop(1 if k == 0 else 0, 64, vfstep, carry)
        vs, p, eacc = carry

        mx = jnp.broadcast_to(jnp.max(vs, axis=0), (16,))
        idx = jnp.broadcast_to(
            plsc.all_reduce_ffs(vs == mx), (16,)).astype(jnp.int32)
        bi_v[T - 1] = idx

        def bstep(q, idx):
            s = (T - 1) - q
            row = bp_v[s]
            ni = row.at[idx].get(mode="promise_in_bounds")
            bi_v[s - 1] = ni
            return ni

        lax.fori_loop(0, T - 1, bstep, idx)

        zeros16 = jnp.zeros((16,), jnp.int32)
        for c in range(T // 16):
            pos = c * 16 + it16
            bi_lane = plsc.load_gather(bi_v, [pos, zeros16])
            pred = plsc.load_gather(bt_v, [pos, bi_lane])
            pred_v[pl.ds(c * 16, 16)] = pred
        pltpu.sync_copy(pred_v, pred_hbm)

        s_tot = jnp.broadcast_to(jnp.sum(p, axis=0), (16,))
        bits = plsc.bitcast(s_tot, jnp.int32)
        e = (((bits >> 23) & 0xFF) - 127).astype(_F32)
        m = plsc.bitcast((bits & 0x7FFFFF) | (127 << 23), _F32)
        y = e * _LN2 + 0.7 * (m - 1.0)
        for _ in range(4):
            y = (y - 1.0) + s_tot * jnp.exp(-y)
        logZ = y + eacc * _LN2
        pltpu.sync_copy(scal_sh, scal2_v)
        base_s = scal2_v[0]
        crf_s = scal2_v[1]
        aux_v[...] = -((base_s + crf_s) - logZ)
        pltpu.sync_copy(aux_v, aux_hbm)


def _sc_crf(scores_flat, tags, crfw_flat, interpret=False):
    mesh = plsc.VectorSubcoreMesh(core_axis_name="c", subcore_axis_name="s",
                                  num_cores=1, num_subcores=_N_TILES)
    f = pl.kernel(
        _sc_body,
        out_type=[jax.ShapeDtypeStruct((T,), jnp.int32),
                  jax.ShapeDtypeStruct((16,), _F32)],
        mesh=mesh,
        compiler_params=pltpu.CompilerParams(needs_layout_passes=False,
                                             use_tc_tiling_on_sc=False),
        scratch_types=[
            pltpu.VMEM((8 * NUM_TAGS,), _F32),
            pltpu.VMEM((_ROWS_PER_TILE, 16), _F32),
            pltpu.VMEM((_ROWS_PER_TILE, 16), jnp.int32),
            pltpu.VMEM((_STEPS_PER_TILE * 256,), jnp.int32),
            pltpu.VMEM((_STEPS_PER_TILE, 256), _F32),
            pltpu.VMEM((_STEPS_PER_TILE, 256), _F32),
            pltpu.VMEM((_STEPS_PER_TILE + 8, 16), jnp.int32),
            pltpu.VMEM((T,), jnp.int32),
            pltpu.VMEM((T,), jnp.int32),
            pltpu.VMEM((T,), _F32),
            pltpu.VMEM((64, 256), _F32),
            pltpu.VMEM((64, 256), _F32),
            pltpu.VMEM((T, 16), _F32),
            pltpu.VMEM((T, 16), jnp.int32),
            pltpu.VMEM((T, 16), jnp.int32),
            pltpu.VMEM((T, 16), jnp.int32),
            pltpu.VMEM((T,), jnp.int32),
            pltpu.VMEM((16,), _F32),
            pltpu.VMEM((8, 16), _F32),
            pltpu.SemaphoreType.DMA,
            pltpu.VMEM_SHARED((T, 16), _F32),
            pltpu.VMEM_SHARED((T, 16), jnp.int32),
            pltpu.VMEM_SHARED((T, 256), _F32),
            pltpu.VMEM_SHARED((T, 256), _F32),
            pltpu.VMEM_SHARED((8, 16), _F32),
        ],
        interpret=interpret,
    )
    scores_flat = pltpu.with_memory_space_constraint(
        scores_flat, pltpu.MemorySpace.HBM)
    tags = pltpu.with_memory_space_constraint(tags, pltpu.MemorySpace.HBM)
    crfw_flat = pltpu.with_memory_space_constraint(
        crfw_flat, pltpu.MemorySpace.HBM)
    return f(scores_flat, tags, crfw_flat)


def kernel(fwd_charIDs, bwd_charIDs, tags, C_emb, Wxf, Whf, bf, Wxb, Whb, bb,
           Wp, bp, Wwf, Uwf, bwf, Wwb, Uwb, bwb, Wo, bo, crf_w):
    fidsT = jnp.transpose(fwd_charIDs).astype(jnp.int32)
    bidsT = jnp.transpose(bwd_charIDs).astype(jnp.int32)
    c_hi = lax.convert_element_type(
        lax.convert_element_type(C_emb, jnp.bfloat16), jnp.float32)
    r1 = C_emb - c_hi
    c_mid = lax.convert_element_type(
        lax.convert_element_type(r1, jnp.bfloat16), jnp.float32)
    c_lo = r1 - c_mid
    c_split = jnp.stack([c_hi, c_mid, c_lo])
    scores = _dense_scores(fidsT, bidsT, c_split, Wxf, Whf, bf[None, :],
                           Wxb, Whb, bb[None, :], Wp, bp[None, :],
                           Wwf, Uwf, bwf[None, :], Wwb, Uwb, bwb[None, :],
                           Wo, bo[None, :])
    pred, aux = _sc_crf(scores.reshape(-1), tags.astype(jnp.int32),
                        crf_w.reshape(-1))
    return pred, aux[0]
